# Initial kernel scaffold; baseline (speedup 1.0000x reference)
import jax
import jax.numpy as jnp
from jax.experimental import pallas as pl


def kernel(x, edge_index, batch, feature_index, threshold, W1, b1, W2, b2):
    raise NotImplementedError("write your pallas kernel here")


# speedup vs baseline: 4.4039x; 4.4039x over previous
import functools

import jax
import jax.numpy as jnp
from jax import lax
from jax.experimental import pallas as pl
from jax.experimental.pallas import tpu as pltpu
from jax.experimental.pallas import tpu_sc as plsc

N_NODES = 100000
B = 2048
D = 128
HIDDEN = 64
OUT_DIM = 2

NC = 2
NS = 16
NW = NC * NS
CHUNK = 160
NCHUNKS = N_NODES // CHUNK
BASE_PER_W = NCHUNKS // NW
EXTRA = NCHUNKS - BASE_PER_W * NW
SEG_PER_TILE = B // NS


def _sc_segment_sum(x, idx):
    mesh = plsc.VectorSubcoreMesh(core_axis_name="c", subcore_axis_name="s")

    @functools.partial(
        pl.kernel,
        mesh=mesh,
        out_type=jax.ShapeDtypeStruct((NC, B, D), jnp.float32),
        scratch_types=[
            pltpu.VMEM((CHUNK, D), jnp.float32),
            pltpu.VMEM((CHUNK,), jnp.int32),
            pltpu.VMEM((SEG_PER_TILE, D), jnp.float32),
            pltpu.VMEM_SHARED((B, D), jnp.float32),
        ],
    )
    def seg_sum(x_hbm, idx_hbm, out_hbm, rows_v, idx_v, zero_v, acc_sh):
        c = lax.axis_index("c")
        s = lax.axis_index("s")
        wid = s * NC + c

        zvec = jnp.zeros((16,), jnp.float32)

        def zero_row(r, _):
            for j in range(D // 16):
                zero_v[r, pl.ds(j * 16, 16)] = zvec
            return 0

        lax.fori_loop(0, SEG_PER_TILE, zero_row, 0)
        pltpu.sync_copy(zero_v, acc_sh.at[pl.ds(s * SEG_PER_TILE, SEG_PER_TILE)])
        plsc.subcore_barrier()

        first = BASE_PER_W * wid + jnp.minimum(wid, EXTRA)
        count = jnp.where(wid < EXTRA, BASE_PER_W + 1, BASE_PER_W)

        def chunk_body(j, _):
            r0 = (first + j) * CHUNK
            pltpu.sync_copy(x_hbm.at[pl.ds(r0, CHUNK)], rows_v)
            pltpu.sync_copy(idx_hbm.at[pl.ds(r0, CHUNK)], idx_v)
            pltpu.sync_copy(rows_v, acc_sh.at[idx_v], add=True)
            return 0

        lax.fori_loop(0, count, chunk_body, 0)
        plsc.subcore_barrier()

        pltpu.sync_copy(
            acc_sh.at[pl.ds(s * SEG_PER_TILE, SEG_PER_TILE)],
            out_hbm.at[c, pl.ds(s * SEG_PER_TILE, SEG_PER_TILE)],
        )

    return seg_sum(x, idx)


def _tc_head(partials, f2d, t2d, W1, b1_2d, W2, b2_2d):
    def head(p_ref, f_ref, t_ref, w1_ref, b1_ref, w2_ref, b2_ref, o_ref):
        xp = p_ref[0] + p_ref[1]
        h = jnp.dot(xp, w1_ref[pl.ds(0, D), :],
                    preferred_element_type=jnp.float32)
        h = h + f_ref[...] * w1_ref[pl.ds(D, 1), :]
        h = h + t_ref[...] * w1_ref[pl.ds(D + 1, 1), :]
        h = h + b1_ref[...]
        h = jnp.where(h >= 0.0, h, 0.01 * h)
        o_ref[...] = jnp.dot(h, w2_ref[...],
                             preferred_element_type=jnp.float32) + b2_ref[...]

    return pl.pallas_call(
        head,
        out_shape=jax.ShapeDtypeStruct((B, OUT_DIM), jnp.float32),
    )(partials, f2d, t2d, W1, b1_2d, W2, b2_2d)


def kernel(x, edge_index, batch, feature_index, threshold, W1, b1, W2, b2):
    partials = _sc_segment_sum(x, batch.astype(jnp.int32))
    out = _tc_head(
        partials,
        feature_index[:, None],
        threshold[:, None],
        W1,
        b1[None, :],
        W2,
        b2[None, :],
    )
    return out


# speedup vs baseline: 6.2000x; 1.4079x over previous
import functools

import jax
import jax.numpy as jnp
from jax import lax
from jax.experimental import pallas as pl
from jax.experimental.pallas import tpu as pltpu
from jax.experimental.pallas import tpu_sc as plsc

N_NODES = 100000
B = 2048
D = 128
HIDDEN = 64
OUT_DIM = 2

NC = 2
NS = 16
NW = NC * NS
CHUNK = 200
NCHUNKS = N_NODES // CHUNK
BASE_PER_W = NCHUNKS // NW
EXTRA = NCHUNKS - BASE_PER_W * NW
SEG_PER_TILE = B // NS


def _sc_segment_sum(x, idx):
    mesh = plsc.VectorSubcoreMesh(core_axis_name="c", subcore_axis_name="s")

    @functools.partial(
        pl.kernel,
        mesh=mesh,
        out_type=jax.ShapeDtypeStruct((NC, B, D), jnp.float32),
        scratch_types=[
            pltpu.VMEM((2, CHUNK, D), jnp.float32),
            pltpu.VMEM((CHUNK,), jnp.int32),
            pltpu.VMEM((CHUNK,), jnp.int32),
            pltpu.VMEM((SEG_PER_TILE, D), jnp.float32),
            pltpu.VMEM_SHARED((B, D), jnp.float32),
            pltpu.SemaphoreType.DMA,
            pltpu.SemaphoreType.DMA,
        ],
    )
    def seg_sum(x_hbm, idx_hbm, out_hbm, rows_v, idx_v0, idx_v1, zero_v,
                acc_sh, sem0, sem1):
        c = lax.axis_index("c")
        s = lax.axis_index("s")
        wid = s * NC + c
        sems = (sem0, sem1)
        idx_bufs = (idx_v0, idx_v1)

        zvec = jnp.zeros((16,), jnp.float32)

        def zero_row(r, _):
            for j in range(D // 16):
                zero_v[r, pl.ds(j * 16, 16)] = zvec
            return 0

        lax.fori_loop(0, SEG_PER_TILE, zero_row, 0)
        pltpu.sync_copy(zero_v, acc_sh.at[pl.ds(s * SEG_PER_TILE, SEG_PER_TILE)])
        plsc.subcore_barrier()

        first = BASE_PER_W * wid + jnp.minimum(wid, EXTRA)
        count = jnp.where(wid < EXTRA, BASE_PER_W + 1, BASE_PER_W)

        def start(j, b):
            r0 = (first + j) * CHUNK
            pltpu.async_copy(x_hbm.at[pl.ds(r0, CHUNK)], rows_v.at[b], sems[b])
            pltpu.async_copy(idx_hbm.at[pl.ds(r0, CHUNK)], idx_bufs[b], sems[b])

        def wait(b):
            pltpu.make_async_copy(
                x_hbm.at[pl.ds(0, CHUNK)], rows_v.at[b], sems[b]).wait()
            pltpu.make_async_copy(
                idx_hbm.at[pl.ds(0, CHUNK)], idx_bufs[b], sems[b]).wait()

        def scatter(b):
            pltpu.sync_copy(rows_v.at[b], acc_sh.at[idx_bufs[b]], add=True)

        start(0, 0)
        pl.when(count > 1)(lambda: start(1, 1))

        def pair_body(i, _):
            j0 = 2 * i

            def do_buf0():
                wait(0)
                scatter(0)
                pl.when(j0 + 2 < count)(lambda: start(j0 + 2, 0))

            def do_buf1():
                wait(1)
                scatter(1)
                pl.when(j0 + 3 < count)(lambda: start(j0 + 3, 1))

            do_buf0()
            pl.when(j0 + 1 < count)(do_buf1)
            return 0

        lax.fori_loop(0, (count + 1) // 2, pair_body, 0)
        plsc.subcore_barrier()

        pltpu.sync_copy(
            acc_sh.at[pl.ds(s * SEG_PER_TILE, SEG_PER_TILE)],
            out_hbm.at[c, pl.ds(s * SEG_PER_TILE, SEG_PER_TILE)],
        )

    return seg_sum(x, idx)


def _tc_head(partials, f2d, t2d, W1, b1_2d, W2, b2_2d):
    def head(p_ref, f_ref, t_ref, w1_ref, b1_ref, w2_ref, b2_ref, o_ref):
        xp = p_ref[0] + p_ref[1]
        h = jnp.dot(xp, w1_ref[pl.ds(0, D), :],
                    preferred_element_type=jnp.float32)
        h = h + f_ref[...] * w1_ref[pl.ds(D, 1), :]
        h = h + t_ref[...] * w1_ref[pl.ds(D + 1, 1), :]
        h = h + b1_ref[...]
        h = jnp.where(h >= 0.0, h, 0.01 * h)
        o_ref[...] = jnp.dot(h, w2_ref[...],
                             preferred_element_type=jnp.float32) + b2_ref[...]

    return pl.pallas_call(
        head,
        out_shape=jax.ShapeDtypeStruct((B, OUT_DIM), jnp.float32),
    )(partials, f2d, t2d, W1, b1_2d, W2, b2_2d)


def kernel(x, edge_index, batch, feature_index, threshold, W1, b1, W2, b2):
    partials = _sc_segment_sum(x, batch.astype(jnp.int32))
    out = _tc_head(
        partials,
        feature_index[:, None],
        threshold[:, None],
        W1,
        b1[None, :],
        W2,
        b2[None, :],
    )
    return out
